# baseline (speedup 1.0000x reference)
import dataclasses
import functools

import jax
import jax.numpy as jnp
from jax import lax
from jax.experimental import pallas as pl
from jax.experimental.pallas import tpu as pltpu
from jax.experimental.pallas import tpu_sc as plsc

N_NODES = 64
N_GROUPS = 100000
BATCH = 16384

NC = 2
NS = 16
NW = NC * NS
LANES = 16

BPW = BATCH // NW
GCH = 128
NG = BPW // GCH


def _sc_dot_kernel(t_hbm, c_hbm, w_hbm, out_hbm, idx_v, rows_v, t_v, buf_v,
                   out_v, gsem, tsem):
    wid = lax.axis_index("s") * NC + lax.axis_index("c")
    base = wid * BPW

    pltpu.sync_copy(c_hbm.at[wid], idx_v)

    t_cp = pltpu.async_copy(t_hbm.at[pl.ds(base, BPW)], t_v, tsem)
    gathers = []
    for g in range(NG):
        gathers.append(
            pltpu.async_copy(w_hbm.at[idx_v.at[g]],
                             rows_v.at[pl.ds(g * GCH, GCH)], gsem))
    t_cp.wait()
    for cp in gathers:
        cp.wait()

    lane_iota = lax.iota(jnp.int32, LANES)

    @pl.loop(0, BPW, step=16)
    def _group(r0):
        for r in range(16):
            row = r0 + r
            acc = rows_v[row, pl.ds(0, LANES)] * t_v[row, pl.ds(0, LANES)]
            for k in range(1, N_NODES // LANES):
                acc = acc + (rows_v[row, pl.ds(k * LANES, LANES)]
                             * t_v[row, pl.ds(k * LANES, LANES)])
            plsc.store_scatter(buf_v, [lane_iota, jnp.full((LANES,), r, jnp.int32)], acc)
        tot = buf_v[0, :]
        for l in range(1, 16):
            tot = tot + buf_v[l, :]
        out_v[pl.ds(r0, 16)] = tot

    pltpu.sync_copy(out_v, out_hbm.at[pl.ds(base, BPW)])


@jax.jit
def kernel(t, c, W):
    c2 = c.reshape(NW, NG, GCH).astype(jnp.int32)
    mesh = plsc.VectorSubcoreMesh(core_axis_name="c", subcore_axis_name="s")
    cp = pltpu.CompilerParams(needs_layout_passes=False,
                              use_tc_tiling_on_sc=False)
    run = functools.partial(
        pl.kernel,
        mesh=mesh,
        compiler_params=cp,
        out_type=jax.ShapeDtypeStruct((BATCH,), jnp.float32),
        scratch_types=[
            pltpu.VMEM((NG, GCH), jnp.int32),
            pltpu.VMEM((BPW, N_NODES), jnp.float32),
            pltpu.VMEM((BPW, N_NODES), jnp.float32),
            pltpu.VMEM((LANES, LANES), jnp.float32),
            pltpu.VMEM((BPW,), jnp.float32),
            pltpu.SemaphoreType.DMA,
            pltpu.SemaphoreType.DMA,
        ],
    )(_sc_dot_kernel)
    return run(t, c2, W)
